# Initial kernel scaffold; baseline (speedup 1.0000x reference)
import jax
import jax.numpy as jnp
from jax.experimental import pallas as pl


def kernel(x, target, text_inputs, eps, params):
    raise NotImplementedError("write your pallas kernel here")


# speedup vs baseline: 1.9058x; 1.9058x over previous
import functools

import jax
import jax.numpy as jnp
from jax import lax
from jax.experimental import pallas as pl
from jax.experimental.pallas import tpu as pltpu
from jax.experimental.pallas import tpu_sc as plsc

VOCAB = 100002
D = 128
B = 4
L = 4096
Y = 50
FILTER_SIZES = [3, 5, 9]
CONV_DIMS = [128, 100, 50]
NFM = 50
LATENT = len(FILTER_SIZES) * NFM // 2
FEAT = len(FILTER_SIZES) * NFM
CP = 128
FEATP = len(FILTER_SIZES) * CP

SC_NC = 2
SC_NS = 16
SC_NW = SC_NC * SC_NS


def _sc_gather(table, idx_flat):
    n = idx_flat.shape[0]
    b_per_w = n // SC_NW
    ch = 128
    nch = b_per_w // ch
    idx2 = idx_flat.reshape(SC_NW * nch, ch)
    mesh = plsc.VectorSubcoreMesh(core_axis_name="c", subcore_axis_name="s")

    @functools.partial(
        pl.kernel,
        mesh=mesh,
        out_type=jax.ShapeDtypeStruct((n, D), jnp.float32),
        scratch_types=[
            pltpu.VMEM((nch, ch), jnp.int32),
            pltpu.VMEM((b_per_w, D), jnp.float32),
            pltpu.SemaphoreType.DMA,
        ],
    )
    def gk(table_hbm, idx_hbm, out_hbm, idx_v, rows_v, sem):
        wid = lax.axis_index("s") * SC_NC + lax.axis_index("c")
        pltpu.sync_copy(idx_hbm.at[pl.ds(wid * nch, nch)], idx_v)
        copies = [
            pltpu.async_copy(
                table_hbm.at[idx_v.at[j]], rows_v.at[pl.ds(j * ch, ch)], sem
            )
            for j in range(nch)
        ]
        for c in copies:
            c.wait()
        pltpu.sync_copy(rows_v, out_hbm.at[pl.ds(wid * b_per_w, b_per_w)])

    return gk(table, idx2)


def _fold_bn(w, g, b, m, v):
    s = g / jnp.sqrt(v + 1e-5)
    return w * s[:, None, None], b - m * s


def _prep_conv(w, bias):
    cout, cin, k = w.shape
    wt = jnp.transpose(w, (2, 1, 0))
    wt = jnp.pad(wt, ((0, 0), (0, CP - cin), (0, CP - cout)))
    bp = jnp.pad(bias, (0, CP - cout)).reshape(1, CP)
    return wt, bp


def _conv(x, w_ref, b, k):
    pad = k // 2
    acc = jnp.dot(x, w_ref[pad], preferred_element_type=jnp.float32)
    for dk in range(k):
        if dk == pad:
            continue
        y = jnp.dot(x, w_ref[dk], preferred_element_type=jnp.float32)
        off = dk - pad
        if off > 0:
            ysh = jnp.concatenate(
                [y[off:], jnp.zeros((off, y.shape[1]), y.dtype)], axis=0
            )
        else:
            ysh = jnp.concatenate(
                [jnp.zeros((-off, y.shape[1]), y.dtype), y[:off]], axis=0
            )
        acc = acc + ysh
    return acc + b


def _tc_kernel_body(refs, *, nweights):
    (emb_ref, tgt_ref, eps_ref), wrefs, (y_ref, bce_ref, kl_ref) = (
        refs[:3], refs[3:3 + nweights], refs[3 + nweights:])
    wi = iter(wrefs)

    def nxt():
        return next(wi)

    x = emb_ref[0]
    res = []
    for k in FILTER_SIZES:
        w0, b0 = nxt(), nxt()[...]
        t = jnp.tanh(_conv(x, w0, b0, k))
        for _blk in range(2):
            w1, b1 = nxt(), nxt()[...]
            w2, b2 = nxt(), nxt()[...]
            ws, bs = nxt()[...], nxt()[...]
            h1 = jnp.tanh(_conv(t, w1, b1, k))
            h2 = _conv(h1, w2, b2, k)
            sc = jnp.dot(t, ws, preferred_element_type=jnp.float32) + bs
            t = jnp.tanh(h2 + sc)
        res.append(t)
    xc = jnp.concatenate(res, axis=1)

    uwt = nxt()[...]
    muw = nxt()[...]
    mub = nxt()[...]
    vaw = nxt()[...]
    vab = nxt()[...]
    fw = nxt()[...]
    fb = nxt()[...]

    scores = jnp.dot(xc, uwt, preferred_element_type=jnp.float32)
    smax = jnp.max(scores, axis=0, keepdims=True)
    e = jnp.exp(scores - smax)
    alpha = e / jnp.sum(e, axis=0, keepdims=True)
    m = lax.dot_general(
        alpha, xc, (((0,), (0,)), ((), ())),
        preferred_element_type=jnp.float32,
    )

    mu = jnp.sum(muw * m, axis=1, keepdims=True) + mub
    lv = jnp.sum(vaw * m, axis=1, keepdims=True) + vab
    epsv = jnp.transpose(eps_ref[0])
    z = epsv * jnp.exp(0.5 * lv) + mu
    yv = jnp.sum(fw * jnp.transpose(z), axis=1, keepdims=True) + fb
    yt = jnp.transpose(yv)
    y_ref[0] = yt

    tgt = tgt_ref[0]
    bce_el = jnp.maximum(yt, 0.0) - yt * tgt + jnp.log1p(jnp.exp(-jnp.abs(yt)))
    bce_ref[0] = jnp.sum(bce_el).reshape(1, 1)
    kl_ref[0] = (-0.5 * jnp.sum(1.0 + lv - mu * mu - jnp.exp(lv))).reshape(1, 1)


def _prep_weights(params):
    ws = []
    for ch in params["channels"]:
        w0, b0 = _prep_conv(ch["base_w"], ch["base_b"])
        ws += [w0, b0]
        for blk in ch["blocks"]:
            w1, bb1 = _fold_bn(blk["w1"], blk["bn1_g"], blk["bn1_b"],
                               blk["bn1_m"], blk["bn1_v"])
            w2, bb2 = _fold_bn(blk["w2"], blk["bn2_g"], blk["bn2_b"],
                               blk["bn2_m"], blk["bn2_v"])
            wsc, bbs = _fold_bn(blk["ws"], blk["bns_g"], blk["bns_b"],
                                blk["bns_m"], blk["bns_v"])
            p1, pb1 = _prep_conv(w1, bb1)
            p2, pb2 = _prep_conv(w2, bb2)
            ps, pbs = _prep_conv(wsc, bbs)
            ws += [p1, pb1, p2, pb2, ps[0], pbs]
    nc = len(FILTER_SIZES)

    def padf(a):
        out = jnp.zeros((a.shape[0], FEATP), a.dtype)
        for c in range(nc):
            out = out.at[:, c * CP:c * CP + NFM].set(a[:, c * NFM:(c + 1) * NFM])
        return out

    ws.append(jnp.transpose(padf(params["U_w"])))
    ws.append(padf(params["mu_w"]))
    ws.append(params["mu_b"].reshape(LATENT, 1))
    ws.append(padf(params["var_w"]))
    ws.append(params["var_b"].reshape(LATENT, 1))
    ws.append(params["final_w"])
    ws.append(params["final_b"].reshape(Y, 1))
    return ws


def _tc_forward(emb, target, eps, weights):
    nw = len(weights)

    def const_spec(a):
        return pl.BlockSpec(a.shape, lambda b: (0,) * a.ndim)

    in_specs = [
        pl.BlockSpec((1, L, D), lambda b: (b, 0, 0)),
        pl.BlockSpec((1, 1, Y), lambda b: (b, 0, 0)),
        pl.BlockSpec((1, 1, LATENT), lambda b: (b, 0, 0)),
    ] + [const_spec(w) for w in weights]
    out_specs = [
        pl.BlockSpec((1, 1, Y), lambda b: (b, 0, 0)),
        pl.BlockSpec((1, 1, 1), lambda b: (b, 0, 0)),
        pl.BlockSpec((1, 1, 1), lambda b: (b, 0, 0)),
    ]
    out_shape = [
        jax.ShapeDtypeStruct((B, 1, Y), jnp.float32),
        jax.ShapeDtypeStruct((B, 1, 1), jnp.float32),
        jax.ShapeDtypeStruct((B, 1, 1), jnp.float32),
    ]

    def body(*refs):
        _tc_kernel_body(refs, nweights=nw)

    y3, bce3, kl3 = pl.pallas_call(
        body,
        grid=(B,),
        in_specs=in_specs,
        out_specs=out_specs,
        out_shape=out_shape,
    )(emb.reshape(B, L, D), target.reshape(B, 1, Y),
      eps.reshape(B, 1, LATENT), *weights)
    return y3, bce3, kl3


def kernel(x, target, text_inputs, eps, params):
    del text_inputs
    emb = _sc_gather(params["embed"], x.reshape(-1))
    weights = _prep_weights(params)
    y3, bce3, kl3 = _tc_forward(emb, target, eps, weights)
    y = y3.reshape(B, Y)
    bce = jnp.sum(bce3) / (B * Y)
    kl = jnp.sum(kl3) / B
    return y, bce, kl


# speedup vs baseline: 1.9268x; 1.0110x over previous
import functools

import jax
import jax.numpy as jnp
from jax import lax
from jax.experimental import pallas as pl
from jax.experimental.pallas import tpu as pltpu
from jax.experimental.pallas import tpu_sc as plsc

VOCAB = 100002
D = 128
B = 4
L = 4096
Y = 50
FILTER_SIZES = [3, 5, 9]
CONV_DIMS = [128, 100, 50]
NFM = 50
LATENT = len(FILTER_SIZES) * NFM // 2
FEAT = len(FILTER_SIZES) * NFM
CP = 128
FEATP = len(FILTER_SIZES) * CP

SC_NC = 2
SC_NS = 16
SC_NW = SC_NC * SC_NS


def _sc_gather(table, idx_flat):
    n = idx_flat.shape[0]
    b_per_w = n // SC_NW
    ch = 128
    nch = b_per_w // ch
    idx2 = idx_flat.reshape(SC_NW * nch, ch)
    mesh = plsc.VectorSubcoreMesh(core_axis_name="c", subcore_axis_name="s")

    @functools.partial(
        pl.kernel,
        mesh=mesh,
        out_type=jax.ShapeDtypeStruct((n, D), jnp.float32),
        scratch_types=[
            pltpu.VMEM((nch, ch), jnp.int32),
            pltpu.VMEM((b_per_w, D), jnp.float32),
            pltpu.SemaphoreType.DMA,
        ],
    )
    def gk(table_hbm, idx_hbm, out_hbm, idx_v, rows_v, sem):
        wid = lax.axis_index("s") * SC_NC + lax.axis_index("c")
        pltpu.sync_copy(idx_hbm.at[pl.ds(wid * nch, nch)], idx_v)
        copies = [
            pltpu.async_copy(
                table_hbm.at[idx_v.at[j]], rows_v.at[pl.ds(j * ch, ch)], sem
            )
            for j in range(nch)
        ]
        for c in copies:
            c.wait()
        pltpu.sync_copy(rows_v, out_hbm.at[pl.ds(wid * b_per_w, b_per_w)])

    return gk(table, idx2)


def _fold_bn(w, g, b, m, v):
    s = g / jnp.sqrt(v + 1e-5)
    return w * s[:, None, None], b - m * s


def _prep_conv(w, bias):
    cout, cin, k = w.shape
    wt = jnp.transpose(w, (2, 1, 0))
    wt = jnp.pad(wt, ((0, 0), (0, CP - cin), (0, CP - cout)))
    bp = jnp.pad(bias, (0, CP - cout)).reshape(1, CP)
    return wt.astype(jnp.bfloat16), bp


def _conv(x, w_ref, b, k):
    pad = k // 2
    xb = x.astype(jnp.bfloat16)
    acc = jnp.dot(xb, w_ref[pad], preferred_element_type=jnp.float32)
    for dk in range(k):
        if dk == pad:
            continue
        y = jnp.dot(xb, w_ref[dk], preferred_element_type=jnp.float32)
        off = dk - pad
        if off > 0:
            ysh = jnp.concatenate(
                [y[off:], jnp.zeros((off, y.shape[1]), y.dtype)], axis=0
            )
        else:
            ysh = jnp.concatenate(
                [jnp.zeros((-off, y.shape[1]), y.dtype), y[:off]], axis=0
            )
        acc = acc + ysh
    return acc + b


def _tc_kernel_body(refs, *, nweights):
    (emb_ref, tgt_ref, eps_ref), wrefs, (y_ref, bce_ref, kl_ref) = (
        refs[:3], refs[3:3 + nweights], refs[3 + nweights:])
    wi = iter(wrefs)

    def nxt():
        return next(wi)

    x = emb_ref[0]
    res = []
    for k in FILTER_SIZES:
        w0, b0 = nxt(), nxt()[...]
        t = jnp.tanh(_conv(x, w0, b0, k))
        for _blk in range(2):
            w1, b1 = nxt(), nxt()[...]
            w2, b2 = nxt(), nxt()[...]
            ws, bs = nxt()[...], nxt()[...]
            h1 = jnp.tanh(_conv(t, w1, b1, k))
            h2 = _conv(h1, w2, b2, k)
            sc = jnp.dot(t.astype(jnp.bfloat16), ws,
                         preferred_element_type=jnp.float32) + bs
            t = jnp.tanh(h2 + sc)
        res.append(t)
    xc = jnp.concatenate(res, axis=1)

    uwt = nxt()[...]
    muw = nxt()[...]
    mub = nxt()[...]
    vaw = nxt()[...]
    vab = nxt()[...]
    fw = nxt()[...]
    fb = nxt()[...]

    xcb = xc.astype(jnp.bfloat16)
    scores = jnp.dot(xcb, uwt, preferred_element_type=jnp.float32)
    smax = jnp.max(scores, axis=0, keepdims=True)
    e = jnp.exp(scores - smax)
    alpha = e / jnp.sum(e, axis=0, keepdims=True)
    m = lax.dot_general(
        alpha.astype(jnp.bfloat16), xcb, (((0,), (0,)), ((), ())),
        preferred_element_type=jnp.float32,
    )

    mu = jnp.sum(muw * m, axis=1, keepdims=True) + mub
    lv = jnp.sum(vaw * m, axis=1, keepdims=True) + vab
    epsv = jnp.transpose(eps_ref[0])
    z = epsv * jnp.exp(0.5 * lv) + mu
    yv = jnp.sum(fw * jnp.transpose(z), axis=1, keepdims=True) + fb
    yt = jnp.transpose(yv)
    y_ref[0] = yt

    tgt = tgt_ref[0]
    bce_el = jnp.maximum(yt, 0.0) - yt * tgt + jnp.log1p(jnp.exp(-jnp.abs(yt)))
    bce_ref[0] = jnp.sum(bce_el).reshape(1, 1)
    kl_ref[0] = (-0.5 * jnp.sum(1.0 + lv - mu * mu - jnp.exp(lv))).reshape(1, 1)


def _prep_weights(params):
    ws = []
    for ch in params["channels"]:
        w0, b0 = _prep_conv(ch["base_w"], ch["base_b"])
        ws += [w0, b0]
        for blk in ch["blocks"]:
            w1, bb1 = _fold_bn(blk["w1"], blk["bn1_g"], blk["bn1_b"],
                               blk["bn1_m"], blk["bn1_v"])
            w2, bb2 = _fold_bn(blk["w2"], blk["bn2_g"], blk["bn2_b"],
                               blk["bn2_m"], blk["bn2_v"])
            wsc, bbs = _fold_bn(blk["ws"], blk["bns_g"], blk["bns_b"],
                                blk["bns_m"], blk["bns_v"])
            p1, pb1 = _prep_conv(w1, bb1)
            p2, pb2 = _prep_conv(w2, bb2)
            ps, pbs = _prep_conv(wsc, bbs)
            ws += [p1, pb1, p2, pb2, ps[0], pbs]
    nc = len(FILTER_SIZES)

    def padf(a):
        out = jnp.zeros((a.shape[0], FEATP), a.dtype)
        for c in range(nc):
            out = out.at[:, c * CP:c * CP + NFM].set(a[:, c * NFM:(c + 1) * NFM])
        return out

    ws.append(jnp.transpose(padf(params["U_w"])).astype(jnp.bfloat16))
    ws.append(padf(params["mu_w"]))
    ws.append(params["mu_b"].reshape(LATENT, 1))
    ws.append(padf(params["var_w"]))
    ws.append(params["var_b"].reshape(LATENT, 1))
    ws.append(params["final_w"])
    ws.append(params["final_b"].reshape(Y, 1))
    return ws


def _tc_forward(emb, target, eps, weights):
    nw = len(weights)

    def const_spec(a):
        return pl.BlockSpec(a.shape, lambda b: (0,) * a.ndim)

    in_specs = [
        pl.BlockSpec((1, L, D), lambda b: (b, 0, 0)),
        pl.BlockSpec((1, 1, Y), lambda b: (b, 0, 0)),
        pl.BlockSpec((1, 1, LATENT), lambda b: (b, 0, 0)),
    ] + [const_spec(w) for w in weights]
    out_specs = [
        pl.BlockSpec((1, 1, Y), lambda b: (b, 0, 0)),
        pl.BlockSpec((1, 1, 1), lambda b: (b, 0, 0)),
        pl.BlockSpec((1, 1, 1), lambda b: (b, 0, 0)),
    ]
    out_shape = [
        jax.ShapeDtypeStruct((B, 1, Y), jnp.float32),
        jax.ShapeDtypeStruct((B, 1, 1), jnp.float32),
        jax.ShapeDtypeStruct((B, 1, 1), jnp.float32),
    ]

    def body(*refs):
        _tc_kernel_body(refs, nweights=nw)

    y3, bce3, kl3 = pl.pallas_call(
        body,
        grid=(B,),
        in_specs=in_specs,
        out_specs=out_specs,
        out_shape=out_shape,
    )(emb.reshape(B, L, D), target.reshape(B, 1, Y),
      eps.reshape(B, 1, LATENT), *weights)
    return y3, bce3, kl3


def kernel(x, target, text_inputs, eps, params):
    del text_inputs
    emb = _sc_gather(params["embed"], x.reshape(-1))
    weights = _prep_weights(params)
    y3, bce3, kl3 = _tc_forward(emb, target, eps, weights)
    y = y3.reshape(B, Y)
    bce = jnp.sum(bce3) / (B * Y)
    kl = jnp.sum(kl3) / B
    return y, bce, kl
